# Initial kernel scaffold; baseline (speedup 1.0000x reference)
import jax
import jax.numpy as jnp
from jax.experimental import pallas as pl


def kernel(x, extra_loss, weights, logits):
    raise NotImplementedError("write your pallas kernel here")


# speedup vs baseline: 1.0264x; 1.0264x over previous
import jax
import jax.numpy as jnp
from jax.experimental import pallas as pl
from jax.experimental.pallas import tpu as pltpu

M = 64
N = 4096
B = 2
F = 2048

NBLK = 256


def _gating_kernel(u_ref, w_ref, l_ref, el_ref, x_ref, out_ref, loss_ref):
    logits = l_ref[...]
    p = jax.nn.sigmoid(logits)
    b = (u_ref[...] < p).astype(jnp.float32)
    scale = jnp.sum(w_ref[...] * b, axis=0)
    log_prob = b * jax.nn.log_sigmoid(logits) + (1.0 - b) * jax.nn.log_sigmoid(-logits)
    loss_ref[...] = el_ref[...] + jnp.sum(log_prob, axis=0, keepdims=True)
    out_ref[...] = x_ref[...] * scale.reshape(1, NBLK, 1)


def kernel(x, extra_loss, weights, logits):
    u = jax.random.uniform(jax.random.key(42), (M, N), jnp.float32)
    el2d = extra_loss.reshape(1, N)
    grid = (N // NBLK,)
    out, loss = pl.pallas_call(
        _gating_kernel,
        grid=grid,
        in_specs=[
            pl.BlockSpec((M, NBLK), lambda i: (0, i)),
            pl.BlockSpec((M, NBLK), lambda i: (0, i)),
            pl.BlockSpec((M, NBLK), lambda i: (0, i)),
            pl.BlockSpec((1, NBLK), lambda i: (0, i)),
            pl.BlockSpec((B, NBLK, F), lambda i: (0, i, 0)),
        ],
        out_specs=[
            pl.BlockSpec((B, NBLK, F), lambda i: (0, i, 0)),
            pl.BlockSpec((1, NBLK), lambda i: (0, i)),
        ],
        out_shape=[
            jax.ShapeDtypeStruct((B, N, F), jnp.float32),
            jax.ShapeDtypeStruct((1, N), jnp.float32),
        ],
        compiler_params=pltpu.CompilerParams(
            dimension_semantics=("arbitrary",),
        ),
    )(u, weights, logits, el2d, x)
    return out, loss.reshape(N)


# speedup vs baseline: 1.0460x; 1.0191x over previous
import jax
import jax.numpy as jnp
from jax.experimental import pallas as pl
from jax.experimental.pallas import tpu as pltpu

M = 64
N = 4096
B = 2
F = 2048

NBLK = 512


def _gating_kernel(u_ref, w_ref, l_ref, el_ref, x_ref, out_ref, loss_ref):
    logits = l_ref[...]
    p = jax.nn.sigmoid(logits)
    b = (u_ref[...] < p).astype(jnp.float32)
    scale = jnp.sum(w_ref[...] * b, axis=0)
    log_prob = b * jax.nn.log_sigmoid(logits) + (1.0 - b) * jax.nn.log_sigmoid(-logits)
    loss_ref[...] = el_ref[...] + jnp.sum(log_prob, axis=0, keepdims=True)
    out_ref[...] = x_ref[...] * scale.reshape(1, NBLK, 1)


def kernel(x, extra_loss, weights, logits):
    u = jax.random.uniform(jax.random.key(42), (M, N), jnp.float32)
    el2d = extra_loss.reshape(1, N)
    grid = (N // NBLK,)
    out, loss = pl.pallas_call(
        _gating_kernel,
        grid=grid,
        in_specs=[
            pl.BlockSpec((M, NBLK), lambda i: (0, i)),
            pl.BlockSpec((M, NBLK), lambda i: (0, i)),
            pl.BlockSpec((M, NBLK), lambda i: (0, i)),
            pl.BlockSpec((1, NBLK), lambda i: (0, i)),
            pl.BlockSpec((B, NBLK, F), lambda i: (0, i, 0)),
        ],
        out_specs=[
            pl.BlockSpec((B, NBLK, F), lambda i: (0, i, 0)),
            pl.BlockSpec((1, NBLK), lambda i: (0, i)),
        ],
        out_shape=[
            jax.ShapeDtypeStruct((B, N, F), jnp.float32),
            jax.ShapeDtypeStruct((1, N), jnp.float32),
        ],
        compiler_params=pltpu.CompilerParams(
            dimension_semantics=("arbitrary",),
        ),
    )(u, weights, logits, el2d, x)
    return out, loss.reshape(N)
